# Initial kernel scaffold; baseline (speedup 1.0000x reference)
import jax
import jax.numpy as jnp
from jax.experimental import pallas as pl


def kernel(x, weight):
    raise NotImplementedError("write your pallas kernel here")


# speedup vs baseline: 2.7655x; 2.7655x over previous
import functools

import jax
import jax.numpy as jnp
from jax import lax
from jax.experimental import pallas as pl
from jax.experimental.pallas import tpu as pltpu, tpu_sc as plsc


def _make_gather(B, D, NC, NS):
    NW = NC * NS
    b_per_w = B // NW
    CHUNK = 128
    n_chunks = b_per_w // CHUNK
    mesh = plsc.VectorSubcoreMesh(core_axis_name="c", subcore_axis_name="s")

    @functools.partial(
        pl.kernel,
        mesh=mesh,
        out_type=jax.ShapeDtypeStruct((B, D), jnp.float32),
        scratch_types=[
            pltpu.VMEM((CHUNK,), jnp.int32),
            pltpu.VMEM((CHUNK, D), jnp.float32),
            pltpu.SemaphoreType.DMA,
        ],
    )
    def k(idx_hbm, table_hbm, out_hbm, idx_v, rows_v, sem):
        wid = lax.axis_index("s") * NC + lax.axis_index("c")
        base = wid * b_per_w

        def body(c, carry):
            off = base + c * CHUNK
            pltpu.sync_copy(idx_hbm.at[pl.ds(off, CHUNK)], idx_v)
            pltpu.async_copy(table_hbm.at[idx_v], rows_v, sem).wait()
            pltpu.sync_copy(rows_v, out_hbm.at[pl.ds(off, CHUNK)])
            return carry

        lax.fori_loop(0, n_chunks, body, 0)

    return k


def kernel(x, weight):
    B = x.shape[0] * x.shape[1]
    D = weight.shape[1]
    idx = x.reshape(B).astype(jnp.int32)
    info = plsc.get_sparse_core_info()
    f = _make_gather(B, D, info.num_cores, info.num_subcores)
    out = f(idx, weight)
    return out.reshape(x.shape + (D,))


# speedup vs baseline: 3.3028x; 1.1943x over previous
import functools

import jax
import jax.numpy as jnp
from jax import lax
from jax.experimental import pallas as pl
from jax.experimental.pallas import tpu as pltpu, tpu_sc as plsc

NBUF = 5


def _make_gather(B, D, NC, NS):
    NW = NC * NS
    b_per_w = B // NW
    CHUNK = 128
    n_chunks = b_per_w // CHUNK
    n_groups = n_chunks // NBUF
    mesh = plsc.VectorSubcoreMesh(core_axis_name="c", subcore_axis_name="s")

    @functools.partial(
        pl.kernel,
        mesh=mesh,
        out_type=jax.ShapeDtypeStruct((B, D), jnp.float32),
        scratch_types=(
            [
                pltpu.VMEM((b_per_w,), jnp.int32),
                pltpu.VMEM((NBUF, CHUNK, D), jnp.float32),
            ]
            + [pltpu.SemaphoreType.DMA] * (2 * NBUF)
        ),
    )
    def k(idx_hbm, table_hbm, out_hbm, idx_v, rows_v, *sems):
        gsem = sems[:NBUF]
        osem = sems[NBUF:]
        wid = lax.axis_index("s") * NC + lax.axis_index("c")
        base = wid * b_per_w

        pltpu.sync_copy(idx_hbm.at[pl.ds(base, b_per_w)], idx_v)

        def start_gather(c, b):
            pltpu.async_copy(
                table_hbm.at[idx_v.at[pl.ds(c * CHUNK, CHUNK)]],
                rows_v.at[b],
                gsem[b],
            )

        def wait_gather(b):
            pltpu.make_async_copy(
                table_hbm.at[idx_v.at[pl.ds(0, CHUNK)]],
                rows_v.at[b],
                gsem[b],
            ).wait()

        def start_store(c, b):
            pltpu.async_copy(
                rows_v.at[b],
                out_hbm.at[pl.ds(base + c * CHUNK, CHUNK)],
                osem[b],
            )

        def wait_store(b):
            pltpu.make_async_copy(
                rows_v.at[b],
                out_hbm.at[pl.ds(base, CHUNK)],
                osem[b],
            ).wait()

        for b in range(NBUF):
            start_gather(b, b)

        def group(g, carry):
            for b in range(NBUF):
                wait_gather(b)
                start_store(g * NBUF + b, b)
            for b in range(NBUF):
                wait_store(b)
                start_gather((g + 1) * NBUF + b, b)
            return carry

        lax.fori_loop(0, n_groups - 1, group, 0)

        last = n_groups - 1
        for b in range(NBUF):
            wait_gather(b)
            start_store(last * NBUF + b, b)
        for b in range(NBUF):
            wait_store(b)

    return k


def kernel(x, weight):
    B = x.shape[0] * x.shape[1]
    D = weight.shape[1]
    idx = x.reshape(B).astype(jnp.int32)
    info = plsc.get_sparse_core_info()
    f = _make_gather(B, D, info.num_cores, info.num_subcores)
    out = f(idx, weight)
    return out.reshape(x.shape + (D,))
